# baseline (device time: 312756 ns/iter reference)
import jax
import jax.numpy as jnp
from jax import lax
from jax.experimental import pallas as pl
from jax.experimental.pallas import tpu as pltpu

N_DEV = 4
M = 2048
N = 2048
M_CH = M // N_DEV


def kernel(x, w_mat):
    k_per = x.shape[1]

    def body(x_ref, w_ref, out_ref, rs_buf, send_sems, recv_sems):
        my = lax.axis_index("i")
        left = (my - 1) % N_DEV
        right = (my + 1) % N_DEV

        barrier_sem = pltpu.get_barrier_semaphore()
        for nbr in (left, right):
            pl.semaphore_signal(
                barrier_sem, inc=1,
                device_id=(nbr,), device_id_type=pl.DeviceIdType.MESH,
            )
        pl.semaphore_wait(barrier_sem, 2)

        out_ref[:, :] = jnp.dot(
            x_ref[:, :], w_ref[:, :], preferred_element_type=jnp.float32
        )

        for s in range(N_DEV - 1):
            send_c = (my - s) % N_DEV
            rdma = pltpu.make_async_remote_copy(
                src_ref=out_ref.at[pl.ds(send_c * M_CH, M_CH), :],
                dst_ref=rs_buf.at[s],
                send_sem=send_sems.at[s],
                recv_sem=recv_sems.at[s],
                device_id=(right,),
                device_id_type=pl.DeviceIdType.MESH,
            )
            rdma.start()
            rdma.wait()
            recv_c = (my - s - 1) % N_DEV
            out_ref[pl.ds(recv_c * M_CH, M_CH), :] += rs_buf[s]

        mc = (my + 1) % N_DEV
        yred = out_ref[pl.ds(mc * M_CH, M_CH), :]
        out_ref[pl.ds(mc * M_CH, M_CH), :] = yred * jax.nn.sigmoid(yred)

        for t in range(N_DEV - 1):
            send_c = (my + 1 - t) % N_DEV
            rdma = pltpu.make_async_remote_copy(
                src_ref=out_ref.at[pl.ds(send_c * M_CH, M_CH), :],
                dst_ref=out_ref.at[pl.ds(send_c * M_CH, M_CH), :],
                send_sem=send_sems.at[N_DEV - 1 + t],
                recv_sem=recv_sems.at[N_DEV - 1 + t],
                device_id=(right,),
                device_id_type=pl.DeviceIdType.MESH,
            )
            rdma.start()
            rdma.wait()

    return pl.pallas_call(
        body,
        out_shape=jax.ShapeDtypeStruct((M, N), jnp.float32),
        in_specs=[
            pl.BlockSpec(memory_space=pltpu.VMEM),
            pl.BlockSpec(memory_space=pltpu.VMEM),
        ],
        out_specs=pl.BlockSpec(memory_space=pltpu.VMEM),
        scratch_shapes=[
            pltpu.VMEM((N_DEV - 1, M_CH, N), jnp.float32),
            pltpu.SemaphoreType.DMA((2 * (N_DEV - 1),)),
            pltpu.SemaphoreType.DMA((2 * (N_DEV - 1),)),
        ],
        compiler_params=pltpu.CompilerParams(collective_id=0),
    )(x, w_mat)


# device time: 178318 ns/iter; 1.7539x vs baseline; 1.7539x over previous
import jax
import jax.numpy as jnp
from jax import lax
from jax.experimental import pallas as pl
from jax.experimental.pallas import tpu as pltpu

N_DEV = 4
M = 2048
N = 2048
M_CH = M // N_DEV
M_H = M_CH // 2


def kernel(x, w_mat):
    def body(x_ref, w_ref, out_ref, buf_cw, buf_ccw,
             send_cw, recv_cw, send_ccw, recv_ccw):
        my = lax.axis_index("i")
        left = (my - 1) % N_DEV
        right = (my + 1) % N_DEV

        barrier_sem = pltpu.get_barrier_semaphore()
        for nbr in (left, right):
            pl.semaphore_signal(
                barrier_sem, inc=1,
                device_id=(nbr,), device_id_type=pl.DeviceIdType.MESH,
            )
        pl.semaphore_wait(barrier_sem, 2)

        out_ref[:, :] = jnp.dot(
            x_ref[:, :], w_ref[:, :], preferred_element_type=jnp.float32
        )

        def a_rows(c):
            return pl.ds(c * M_CH, M_H)

        def b_rows(c):
            return pl.ds(c * M_CH + M_H, M_H)

        for s in range(N_DEV - 1):
            cw = pltpu.make_async_remote_copy(
                src_ref=out_ref.at[a_rows((my - s) % N_DEV), :],
                dst_ref=buf_cw.at[s],
                send_sem=send_cw.at[s],
                recv_sem=recv_cw.at[s],
                device_id=(right,),
                device_id_type=pl.DeviceIdType.MESH,
            )
            ccw = pltpu.make_async_remote_copy(
                src_ref=out_ref.at[b_rows((my + s) % N_DEV), :],
                dst_ref=buf_ccw.at[s],
                send_sem=send_ccw.at[s],
                recv_sem=recv_ccw.at[s],
                device_id=(left,),
                device_id_type=pl.DeviceIdType.MESH,
            )
            cw.start()
            ccw.start()
            cw.wait()
            ccw.wait()
            out_ref[a_rows((my - s - 1) % N_DEV), :] += buf_cw[s]
            out_ref[b_rows((my + s + 1) % N_DEV), :] += buf_ccw[s]

        ca = (my + 1) % N_DEV
        cb = (my - 1) % N_DEV
        ya = out_ref[a_rows(ca), :]
        out_ref[a_rows(ca), :] = ya * jax.nn.sigmoid(ya)
        yb = out_ref[b_rows(cb), :]
        out_ref[b_rows(cb), :] = yb * jax.nn.sigmoid(yb)

        for t in range(N_DEV - 1):
            sa = (my + 1 - t) % N_DEV
            sb = (my - 1 + t) % N_DEV
            cw = pltpu.make_async_remote_copy(
                src_ref=out_ref.at[a_rows(sa), :],
                dst_ref=out_ref.at[a_rows(sa), :],
                send_sem=send_cw.at[N_DEV - 1 + t],
                recv_sem=recv_cw.at[N_DEV - 1 + t],
                device_id=(right,),
                device_id_type=pl.DeviceIdType.MESH,
            )
            ccw = pltpu.make_async_remote_copy(
                src_ref=out_ref.at[b_rows(sb), :],
                dst_ref=out_ref.at[b_rows(sb), :],
                send_sem=send_ccw.at[N_DEV - 1 + t],
                recv_sem=recv_ccw.at[N_DEV - 1 + t],
                device_id=(left,),
                device_id_type=pl.DeviceIdType.MESH,
            )
            cw.start()
            ccw.start()
            cw.wait()
            ccw.wait()

    n_hops = 2 * (N_DEV - 1)
    return pl.pallas_call(
        body,
        out_shape=jax.ShapeDtypeStruct((M, N), jnp.float32),
        in_specs=[
            pl.BlockSpec(memory_space=pltpu.VMEM),
            pl.BlockSpec(memory_space=pltpu.VMEM),
        ],
        out_specs=pl.BlockSpec(memory_space=pltpu.VMEM),
        scratch_shapes=[
            pltpu.VMEM((N_DEV - 1, M_H, N), jnp.float32),
            pltpu.VMEM((N_DEV - 1, M_H, N), jnp.float32),
            pltpu.SemaphoreType.DMA((n_hops,)),
            pltpu.SemaphoreType.DMA((n_hops,)),
            pltpu.SemaphoreType.DMA((n_hops,)),
            pltpu.SemaphoreType.DMA((n_hops,)),
        ],
        compiler_params=pltpu.CompilerParams(collective_id=0),
    )(x, w_mat)


# device time: 174197 ns/iter; 1.7954x vs baseline; 1.0237x over previous
import jax
import jax.numpy as jnp
from jax import lax
from jax.experimental import pallas as pl
from jax.experimental.pallas import tpu as pltpu

N_DEV = 4
M = 2048
N = 2048
M_CH = M // N_DEV
M_H = M_CH // 2


def kernel(x, w_mat):
    def body(x_ref, w_ref, out_ref, buf_cw, buf_ccw,
             send_cw, recv_cw, send_ccw, recv_ccw):
        my = lax.axis_index("i")
        left = (my - 1) % N_DEV
        right = (my + 1) % N_DEV

        barrier_sem = pltpu.get_barrier_semaphore()
        for nbr in (left, right):
            pl.semaphore_signal(
                barrier_sem, inc=1,
                device_id=(nbr,), device_id_type=pl.DeviceIdType.MESH,
            )
        pl.semaphore_wait(barrier_sem, 2)

        def a_rows(c):
            return pl.ds(c * M_CH, M_H)

        def b_rows(c):
            return pl.ds(c * M_CH + M_H, M_H)

        def ch_rows(c):
            return pl.ds(c * M_CH, M_CH)

        def rs_cw(s):
            return pltpu.make_async_remote_copy(
                src_ref=out_ref.at[a_rows((my - s) % N_DEV), :],
                dst_ref=buf_cw.at[s],
                send_sem=send_cw.at[s],
                recv_sem=recv_cw.at[s],
                device_id=(right,),
                device_id_type=pl.DeviceIdType.MESH,
            )

        def rs_ccw(s):
            return pltpu.make_async_remote_copy(
                src_ref=out_ref.at[b_rows((my + s) % N_DEV), :],
                dst_ref=buf_ccw.at[s],
                send_sem=send_ccw.at[s],
                recv_sem=recv_ccw.at[s],
                device_id=(left,),
                device_id_type=pl.DeviceIdType.MESH,
            )

        def ag_cw(t):
            c = (my + 1 - t) % N_DEV
            return pltpu.make_async_remote_copy(
                src_ref=out_ref.at[a_rows(c), :],
                dst_ref=out_ref.at[a_rows(c), :],
                send_sem=send_cw.at[N_DEV - 1 + t],
                recv_sem=recv_cw.at[N_DEV - 1 + t],
                device_id=(right,),
                device_id_type=pl.DeviceIdType.MESH,
            )

        def ag_ccw(t):
            c = (my - 1 + t) % N_DEV
            return pltpu.make_async_remote_copy(
                src_ref=out_ref.at[b_rows(c), :],
                dst_ref=out_ref.at[b_rows(c), :],
                send_sem=send_ccw.at[N_DEV - 1 + t],
                recv_sem=recv_ccw.at[N_DEV - 1 + t],
                device_id=(left,),
                device_id_type=pl.DeviceIdType.MESH,
            )

        pending_sends = []

        c0 = my % N_DEV
        out_ref[ch_rows(c0), :] = jnp.dot(
            x_ref[ch_rows(c0), :], w_ref[:, :],
            preferred_element_type=jnp.float32,
        )
        cw = rs_cw(0)
        ccw = rs_ccw(0)
        cw.start()
        ccw.start()
        pending_sends += [cw, ccw]

        for o in range(1, N_DEV):
            c = (my + o) % N_DEV
            out_ref[ch_rows(c), :] = jnp.dot(
                x_ref[ch_rows(c), :], w_ref[:, :],
                preferred_element_type=jnp.float32,
            )

        for s in range(N_DEV - 1):
            cw_cur, ccw_cur = cw, ccw
            cw_cur.wait_recv()
            out_ref[a_rows((my - s - 1) % N_DEV), :] += buf_cw[s]
            if s < N_DEV - 2:
                cw = rs_cw(s + 1)
                cw.start()
                pending_sends.append(cw)
            ccw_cur.wait_recv()
            out_ref[b_rows((my + s + 1) % N_DEV), :] += buf_ccw[s]
            if s < N_DEV - 2:
                ccw = rs_ccw(s + 1)
                ccw.start()
                pending_sends.append(ccw)

        ca = (my + 1) % N_DEV
        ya = out_ref[a_rows(ca), :]
        out_ref[a_rows(ca), :] = ya * jax.nn.sigmoid(ya)
        cw = ag_cw(0)
        cw.start()
        pending_sends.append(cw)

        cb = (my - 1) % N_DEV
        yb = out_ref[b_rows(cb), :]
        out_ref[b_rows(cb), :] = yb * jax.nn.sigmoid(yb)
        ccw = ag_ccw(0)
        ccw.start()
        pending_sends.append(ccw)

        for t in range(N_DEV - 1):
            cw_cur, ccw_cur = cw, ccw
            cw_cur.wait_recv()
            if t < N_DEV - 2:
                cw = ag_cw(t + 1)
                cw.start()
                pending_sends.append(cw)
            ccw_cur.wait_recv()
            if t < N_DEV - 2:
                ccw = ag_ccw(t + 1)
                ccw.start()
                pending_sends.append(ccw)

        for d in pending_sends:
            d.wait_send()

    n_hops = 2 * (N_DEV - 1)
    return pl.pallas_call(
        body,
        out_shape=jax.ShapeDtypeStruct((M, N), jnp.float32),
        in_specs=[
            pl.BlockSpec(memory_space=pltpu.VMEM),
            pl.BlockSpec(memory_space=pltpu.VMEM),
        ],
        out_specs=pl.BlockSpec(memory_space=pltpu.VMEM),
        scratch_shapes=[
            pltpu.VMEM((N_DEV - 1, M_H, N), jnp.float32),
            pltpu.VMEM((N_DEV - 1, M_H, N), jnp.float32),
            pltpu.SemaphoreType.DMA((n_hops,)),
            pltpu.SemaphoreType.DMA((n_hops,)),
            pltpu.SemaphoreType.DMA((n_hops,)),
            pltpu.SemaphoreType.DMA((n_hops,)),
        ],
        compiler_params=pltpu.CompilerParams(collective_id=0),
    )(x, w_mat)


# device time: 163204 ns/iter; 1.9164x vs baseline; 1.0674x over previous
import jax
import jax.numpy as jnp
from jax import lax
from jax.experimental import pallas as pl
from jax.experimental.pallas import tpu as pltpu

N_DEV = 4
M = 2048
N = 2048
M_CH = M // N_DEV
M_H = M_CH // 2
K_SUB = 2
M_S = M_H // K_SUB
N_HOP = N_DEV - 1


def kernel(x, w_mat):
    def body(x_ref, w_ref, out_ref, buf_cw, buf_ccw,
             send_cw, recv_cw, send_ccw, recv_ccw):
        my = lax.axis_index("i")
        left = (my - 1) % N_DEV
        right = (my + 1) % N_DEV

        barrier_sem = pltpu.get_barrier_semaphore()
        for nbr in (left, right):
            pl.semaphore_signal(
                barrier_sem, inc=1,
                device_id=(nbr,), device_id_type=pl.DeviceIdType.MESH,
            )
        pl.semaphore_wait(barrier_sem, 2)

        def a_rows(c, j):
            return pl.ds(c * M_CH + j * M_S, M_S)

        def b_rows(c, j):
            return pl.ds(c * M_CH + M_H + j * M_S, M_S)

        def ch_rows(c):
            return pl.ds(c * M_CH, M_CH)

        def rs_cw(s, j):
            return pltpu.make_async_remote_copy(
                src_ref=out_ref.at[a_rows((my - s) % N_DEV, j), :],
                dst_ref=buf_cw.at[s * K_SUB + j],
                send_sem=send_cw.at[s * K_SUB + j],
                recv_sem=recv_cw.at[s * K_SUB + j],
                device_id=(right,),
                device_id_type=pl.DeviceIdType.MESH,
            )

        def rs_ccw(s, j):
            return pltpu.make_async_remote_copy(
                src_ref=out_ref.at[b_rows((my + s) % N_DEV, j), :],
                dst_ref=buf_ccw.at[s * K_SUB + j],
                send_sem=send_ccw.at[s * K_SUB + j],
                recv_sem=recv_ccw.at[s * K_SUB + j],
                device_id=(left,),
                device_id_type=pl.DeviceIdType.MESH,
            )

        def ag_cw(t, j):
            c = (my + 1 - t) % N_DEV
            return pltpu.make_async_remote_copy(
                src_ref=out_ref.at[a_rows(c, j), :],
                dst_ref=out_ref.at[a_rows(c, j), :],
                send_sem=send_cw.at[(N_HOP + t) * K_SUB + j],
                recv_sem=recv_cw.at[(N_HOP + t) * K_SUB + j],
                device_id=(right,),
                device_id_type=pl.DeviceIdType.MESH,
            )

        def ag_ccw(t, j):
            c = (my - 1 + t) % N_DEV
            return pltpu.make_async_remote_copy(
                src_ref=out_ref.at[b_rows(c, j), :],
                dst_ref=out_ref.at[b_rows(c, j), :],
                send_sem=send_ccw.at[(N_HOP + t) * K_SUB + j],
                recv_sem=recv_ccw.at[(N_HOP + t) * K_SUB + j],
                device_id=(left,),
                device_id_type=pl.DeviceIdType.MESH,
            )

        pending_sends = []

        def launch(d):
            d.start()
            pending_sends.append(d)
            return d

        c0 = my % N_DEV
        out_ref[ch_rows(c0), :] = jnp.dot(
            x_ref[ch_rows(c0), :], w_ref[:, :],
            preferred_element_type=jnp.float32,
        )
        cw = [launch(rs_cw(0, j)) for j in range(K_SUB)]
        ccw = [launch(rs_ccw(0, j)) for j in range(K_SUB)]

        for o in range(1, N_DEV):
            c = (my + o) % N_DEV
            out_ref[ch_rows(c), :] = jnp.dot(
                x_ref[ch_rows(c), :], w_ref[:, :],
                preferred_element_type=jnp.float32,
            )

        for s in range(N_HOP - 1):
            for j in range(K_SUB):
                cw[j].wait_recv()
                out_ref[a_rows((my - s - 1) % N_DEV, j), :] += buf_cw[s * K_SUB + j]
                cw[j] = launch(rs_cw(s + 1, j))
                ccw[j].wait_recv()
                out_ref[b_rows((my + s + 1) % N_DEV, j), :] += buf_ccw[s * K_SUB + j]
                ccw[j] = launch(rs_ccw(s + 1, j))

        s = N_HOP - 1
        ca = (my + 1) % N_DEV
        cb = (my - 1) % N_DEV
        for j in range(K_SUB):
            cw[j].wait_recv()
            ya = out_ref[a_rows(ca, j), :] + buf_cw[s * K_SUB + j]
            out_ref[a_rows(ca, j), :] = ya * jax.nn.sigmoid(ya)
            cw[j] = launch(ag_cw(0, j))
            ccw[j].wait_recv()
            yb = out_ref[b_rows(cb, j), :] + buf_ccw[s * K_SUB + j]
            out_ref[b_rows(cb, j), :] = yb * jax.nn.sigmoid(yb)
            ccw[j] = launch(ag_ccw(0, j))

        for t in range(N_HOP):
            for j in range(K_SUB):
                cw[j].wait_recv()
                if t < N_HOP - 1:
                    cw[j] = launch(ag_cw(t + 1, j))
                ccw[j].wait_recv()
                if t < N_HOP - 1:
                    ccw[j] = launch(ag_ccw(t + 1, j))

        for d in pending_sends:
            d.wait_send()

    n_slots = 2 * N_HOP * K_SUB
    return pl.pallas_call(
        body,
        out_shape=jax.ShapeDtypeStruct((M, N), jnp.float32),
        in_specs=[
            pl.BlockSpec(memory_space=pltpu.VMEM),
            pl.BlockSpec(memory_space=pltpu.VMEM),
        ],
        out_specs=pl.BlockSpec(memory_space=pltpu.VMEM),
        scratch_shapes=[
            pltpu.VMEM((N_HOP * K_SUB, M_S, N), jnp.float32),
            pltpu.VMEM((N_HOP * K_SUB, M_S, N), jnp.float32),
            pltpu.SemaphoreType.DMA((n_slots,)),
            pltpu.SemaphoreType.DMA((n_slots,)),
            pltpu.SemaphoreType.DMA((n_slots,)),
            pltpu.SemaphoreType.DMA((n_slots,)),
        ],
        compiler_params=pltpu.CompilerParams(collective_id=0),
    )(x, w_mat)


# device time: 159672 ns/iter; 1.9587x vs baseline; 1.0221x over previous
import jax
import jax.numpy as jnp
from jax import lax
from jax.experimental import pallas as pl
from jax.experimental.pallas import tpu as pltpu

N_DEV = 4
M = 2048
N = 2048
M_CH = M // N_DEV
M_H = M_CH // 2
K_SUB = 2
M_S = M_H // K_SUB
N_HOP = N_DEV - 1


def kernel(x, w_mat):
    def body(x_ref, w_ref, out_hbm, y_ref, buf_cw, buf_ccw,
             send_cw, recv_cw, send_ccw, recv_ccw, store_sems):
        my = lax.axis_index("i")
        left = (my - 1) % N_DEV
        right = (my + 1) % N_DEV

        barrier_sem = pltpu.get_barrier_semaphore()
        for nbr in (left, right):
            pl.semaphore_signal(
                barrier_sem, inc=1,
                device_id=(nbr,), device_id_type=pl.DeviceIdType.MESH,
            )
        pl.semaphore_wait(barrier_sem, 2)

        def a_rows(c, j):
            return pl.ds(c * M_CH + j * M_S, M_S)

        def b_rows(c, j):
            return pl.ds(c * M_CH + M_H + j * M_S, M_S)

        def ch_rows(c):
            return pl.ds(c * M_CH, M_CH)

        def rs_cw(s, j):
            return pltpu.make_async_remote_copy(
                src_ref=y_ref.at[a_rows((my - s) % N_DEV, j), :],
                dst_ref=buf_cw.at[s * K_SUB + j],
                send_sem=send_cw.at[s * K_SUB + j],
                recv_sem=recv_cw.at[s * K_SUB + j],
                device_id=(right,),
                device_id_type=pl.DeviceIdType.MESH,
            )

        def rs_ccw(s, j):
            return pltpu.make_async_remote_copy(
                src_ref=y_ref.at[b_rows((my + s) % N_DEV, j), :],
                dst_ref=buf_ccw.at[s * K_SUB + j],
                send_sem=send_ccw.at[s * K_SUB + j],
                recv_sem=recv_ccw.at[s * K_SUB + j],
                device_id=(left,),
                device_id_type=pl.DeviceIdType.MESH,
            )

        def ag_cw(t, j):
            c = (my + 1 - t) % N_DEV
            return pltpu.make_async_remote_copy(
                src_ref=y_ref.at[a_rows(c, j), :],
                dst_ref=y_ref.at[a_rows(c, j), :],
                send_sem=send_cw.at[(N_HOP + t) * K_SUB + j],
                recv_sem=recv_cw.at[(N_HOP + t) * K_SUB + j],
                device_id=(right,),
                device_id_type=pl.DeviceIdType.MESH,
            )

        def ag_ccw(t, j):
            c = (my - 1 + t) % N_DEV
            return pltpu.make_async_remote_copy(
                src_ref=y_ref.at[b_rows(c, j), :],
                dst_ref=y_ref.at[b_rows(c, j), :],
                send_sem=send_ccw.at[(N_HOP + t) * K_SUB + j],
                recv_sem=recv_ccw.at[(N_HOP + t) * K_SUB + j],
                device_id=(left,),
                device_id_type=pl.DeviceIdType.MESH,
            )

        pending_sends = []
        pending_stores = []
        n_stores = [0]

        def launch(d):
            d.start()
            pending_sends.append(d)
            return d

        def store(rows):
            cp = pltpu.make_async_copy(
                y_ref.at[rows, :], out_hbm.at[rows, :],
                store_sems.at[n_stores[0]],
            )
            n_stores[0] += 1
            cp.start()
            pending_stores.append(cp)

        cw = [None] * K_SUB
        ccw = [None] * K_SUB
        for j in range(K_SUB):
            r = a_rows(my, j)
            y_ref[r, :] = jnp.dot(
                x_ref[r, :], w_ref[:, :], preferred_element_type=jnp.float32
            )
            cw[j] = launch(rs_cw(0, j))
        for j in range(K_SUB):
            r = b_rows(my, j)
            y_ref[r, :] = jnp.dot(
                x_ref[r, :], w_ref[:, :], preferred_element_type=jnp.float32
            )
            ccw[j] = launch(rs_ccw(0, j))

        for o in range(1, N_DEV):
            c = (my + o) % N_DEV
            y_ref[ch_rows(c), :] = jnp.dot(
                x_ref[ch_rows(c), :], w_ref[:, :],
                preferred_element_type=jnp.float32,
            )

        for s in range(N_HOP - 1):
            for j in range(K_SUB):
                cw[j].wait_recv()
                y_ref[a_rows((my - s - 1) % N_DEV, j), :] += buf_cw[s * K_SUB + j]
                cw[j] = launch(rs_cw(s + 1, j))
                ccw[j].wait_recv()
                y_ref[b_rows((my + s + 1) % N_DEV, j), :] += buf_ccw[s * K_SUB + j]
                ccw[j] = launch(rs_ccw(s + 1, j))

        s = N_HOP - 1
        ca = (my + 1) % N_DEV
        cb = (my - 1) % N_DEV
        for j in range(K_SUB):
            cw[j].wait_recv()
            ya = y_ref[a_rows(ca, j), :] + buf_cw[s * K_SUB + j]
            y_ref[a_rows(ca, j), :] = ya * jax.nn.sigmoid(ya)
            cw[j] = launch(ag_cw(0, j))
            store(a_rows(ca, j))
            ccw[j].wait_recv()
            yb = y_ref[b_rows(cb, j), :] + buf_ccw[s * K_SUB + j]
            y_ref[b_rows(cb, j), :] = yb * jax.nn.sigmoid(yb)
            ccw[j] = launch(ag_ccw(0, j))
            store(b_rows(cb, j))

        for t in range(N_HOP):
            for j in range(K_SUB):
                cw[j].wait_recv()
                if t < N_HOP - 1:
                    cw[j] = launch(ag_cw(t + 1, j))
                store(a_rows((my - t) % N_DEV, j))
                ccw[j].wait_recv()
                if t < N_HOP - 1:
                    ccw[j] = launch(ag_ccw(t + 1, j))
                store(b_rows((my + t) % N_DEV, j))

        for d in pending_sends:
            d.wait_send()
        for cp in pending_stores:
            cp.wait()

    n_slots = 2 * N_HOP * K_SUB
    n_store_slots = 2 * N_DEV * K_SUB
    return pl.pallas_call(
        body,
        out_shape=jax.ShapeDtypeStruct((M, N), jnp.float32),
        in_specs=[
            pl.BlockSpec(memory_space=pltpu.VMEM),
            pl.BlockSpec(memory_space=pltpu.VMEM),
        ],
        out_specs=pl.BlockSpec(memory_space=pl.MemorySpace.ANY),
        scratch_shapes=[
            pltpu.VMEM((M, N), jnp.float32),
            pltpu.VMEM((N_HOP * K_SUB, M_S, N), jnp.float32),
            pltpu.VMEM((N_HOP * K_SUB, M_S, N), jnp.float32),
            pltpu.SemaphoreType.DMA((n_slots,)),
            pltpu.SemaphoreType.DMA((n_slots,)),
            pltpu.SemaphoreType.DMA((n_slots,)),
            pltpu.SemaphoreType.DMA((n_slots,)),
            pltpu.SemaphoreType.DMA((n_store_slots,)),
        ],
        compiler_params=pltpu.CompilerParams(collective_id=0),
    )(x, w_mat)


# device time: 156837 ns/iter; 1.9941x vs baseline; 1.0181x over previous
import jax
import jax.numpy as jnp
from jax import lax
from jax.experimental import pallas as pl
from jax.experimental.pallas import tpu as pltpu

N_DEV = 4
M = 2048
N = 2048
M_CH = M // N_DEV
M_H = M_CH // 2
K_SUB = 2
M_S = M_H // K_SUB
N_HOP = N_DEV - 1


def kernel(x, w_mat):
    def body(x_hbm, w_hbm, out_hbm, x_ref, w_ref, y_ref, buf_cw, buf_ccw,
             send_cw, recv_cw, send_ccw, recv_ccw, store_sems, load_sems):
        my = lax.axis_index("i")
        left = (my - 1) % N_DEV
        right = (my + 1) % N_DEV

        w_load = pltpu.make_async_copy(w_hbm, w_ref, load_sems.at[0])
        w_load.start()
        x_loads = []
        for o in range(N_DEV):
            c = (my + o) % N_DEV
            r = pl.ds(c * M_CH, M_CH)
            cp = pltpu.make_async_copy(
                x_hbm.at[r, :], x_ref.at[r, :], load_sems.at[1 + o]
            )
            cp.start()
            x_loads.append(cp)

        barrier_sem = pltpu.get_barrier_semaphore()
        for nbr in (left, right):
            pl.semaphore_signal(
                barrier_sem, inc=1,
                device_id=(nbr,), device_id_type=pl.DeviceIdType.MESH,
            )
        pl.semaphore_wait(barrier_sem, 2)

        def a_rows(c, j):
            return pl.ds(c * M_CH + j * M_S, M_S)

        def b_rows(c, j):
            return pl.ds(c * M_CH + M_H + j * M_S, M_S)

        def ch_rows(c):
            return pl.ds(c * M_CH, M_CH)

        def rs_cw(s, j):
            return pltpu.make_async_remote_copy(
                src_ref=y_ref.at[a_rows((my - s) % N_DEV, j), :],
                dst_ref=buf_cw.at[s * K_SUB + j],
                send_sem=send_cw.at[s * K_SUB + j],
                recv_sem=recv_cw.at[s * K_SUB + j],
                device_id=(right,),
                device_id_type=pl.DeviceIdType.MESH,
            )

        def rs_ccw(s, j):
            return pltpu.make_async_remote_copy(
                src_ref=y_ref.at[b_rows((my + s) % N_DEV, j), :],
                dst_ref=buf_ccw.at[s * K_SUB + j],
                send_sem=send_ccw.at[s * K_SUB + j],
                recv_sem=recv_ccw.at[s * K_SUB + j],
                device_id=(left,),
                device_id_type=pl.DeviceIdType.MESH,
            )

        def ag_cw(t, j):
            c = (my + 1 - t) % N_DEV
            return pltpu.make_async_remote_copy(
                src_ref=y_ref.at[a_rows(c, j), :],
                dst_ref=y_ref.at[a_rows(c, j), :],
                send_sem=send_cw.at[(N_HOP + t) * K_SUB + j],
                recv_sem=recv_cw.at[(N_HOP + t) * K_SUB + j],
                device_id=(right,),
                device_id_type=pl.DeviceIdType.MESH,
            )

        def ag_ccw(t, j):
            c = (my - 1 + t) % N_DEV
            return pltpu.make_async_remote_copy(
                src_ref=y_ref.at[b_rows(c, j), :],
                dst_ref=y_ref.at[b_rows(c, j), :],
                send_sem=send_ccw.at[(N_HOP + t) * K_SUB + j],
                recv_sem=recv_ccw.at[(N_HOP + t) * K_SUB + j],
                device_id=(left,),
                device_id_type=pl.DeviceIdType.MESH,
            )

        pending_sends = []
        pending_stores = []
        n_stores = [0]

        def launch(d):
            d.start()
            pending_sends.append(d)
            return d

        def store(rows):
            cp = pltpu.make_async_copy(
                y_ref.at[rows, :], out_hbm.at[rows, :],
                store_sems.at[n_stores[0]],
            )
            n_stores[0] += 1
            cp.start()
            pending_stores.append(cp)

        w_load.wait()
        x_loads[0].wait()
        cw = [None] * K_SUB
        ccw = [None] * K_SUB
        for j in range(K_SUB):
            r = a_rows(my, j)
            y_ref[r, :] = jnp.dot(
                x_ref[r, :], w_ref[:, :], preferred_element_type=jnp.float32
            )
            cw[j] = launch(rs_cw(0, j))
        for j in range(K_SUB):
            r = b_rows(my, j)
            y_ref[r, :] = jnp.dot(
                x_ref[r, :], w_ref[:, :], preferred_element_type=jnp.float32
            )
            ccw[j] = launch(rs_ccw(0, j))

        for o in range(1, N_DEV):
            c = (my + o) % N_DEV
            x_loads[o].wait()
            y_ref[ch_rows(c), :] = jnp.dot(
                x_ref[ch_rows(c), :], w_ref[:, :],
                preferred_element_type=jnp.float32,
            )

        for s in range(N_HOP - 1):
            for j in range(K_SUB):
                cw[j].wait_recv()
                y_ref[a_rows((my - s - 1) % N_DEV, j), :] += buf_cw[s * K_SUB + j]
                cw[j] = launch(rs_cw(s + 1, j))
                ccw[j].wait_recv()
                y_ref[b_rows((my + s + 1) % N_DEV, j), :] += buf_ccw[s * K_SUB + j]
                ccw[j] = launch(rs_ccw(s + 1, j))

        s = N_HOP - 1
        ca = (my + 1) % N_DEV
        cb = (my - 1) % N_DEV
        for j in range(K_SUB):
            cw[j].wait_recv()
            ya = y_ref[a_rows(ca, j), :] + buf_cw[s * K_SUB + j]
            y_ref[a_rows(ca, j), :] = ya * jax.nn.sigmoid(ya)
            cw[j] = launch(ag_cw(0, j))
            store(a_rows(ca, j))
            ccw[j].wait_recv()
            yb = y_ref[b_rows(cb, j), :] + buf_ccw[s * K_SUB + j]
            y_ref[b_rows(cb, j), :] = yb * jax.nn.sigmoid(yb)
            ccw[j] = launch(ag_ccw(0, j))
            store(b_rows(cb, j))

        for t in range(N_HOP):
            for j in range(K_SUB):
                cw[j].wait_recv()
                if t < N_HOP - 1:
                    cw[j] = launch(ag_cw(t + 1, j))
                store(a_rows((my - t) % N_DEV, j))
                ccw[j].wait_recv()
                if t < N_HOP - 1:
                    ccw[j] = launch(ag_ccw(t + 1, j))
                store(b_rows((my + t) % N_DEV, j))

        for d in pending_sends:
            d.wait_send()
        for cp in pending_stores:
            cp.wait()

    n_slots = 2 * N_HOP * K_SUB
    n_store_slots = 2 * N_DEV * K_SUB
    return pl.pallas_call(
        body,
        out_shape=jax.ShapeDtypeStruct((M, N), jnp.float32),
        in_specs=[
            pl.BlockSpec(memory_space=pl.MemorySpace.ANY),
            pl.BlockSpec(memory_space=pl.MemorySpace.ANY),
        ],
        out_specs=pl.BlockSpec(memory_space=pl.MemorySpace.ANY),
        scratch_shapes=[
            pltpu.VMEM((M, x.shape[1]), jnp.float32),
            pltpu.VMEM((x.shape[1], N), jnp.float32),
            pltpu.VMEM((M, N), jnp.float32),
            pltpu.VMEM((N_HOP * K_SUB, M_S, N), jnp.float32),
            pltpu.VMEM((N_HOP * K_SUB, M_S, N), jnp.float32),
            pltpu.SemaphoreType.DMA((n_slots,)),
            pltpu.SemaphoreType.DMA((n_slots,)),
            pltpu.SemaphoreType.DMA((n_slots,)),
            pltpu.SemaphoreType.DMA((n_slots,)),
            pltpu.SemaphoreType.DMA((n_store_slots,)),
            pltpu.SemaphoreType.DMA((1 + N_DEV,)),
        ],
        compiler_params=pltpu.CompilerParams(
            collective_id=0,
            vmem_limit_bytes=64 * 1024 * 1024,
        ),
    )(x, w_mat)


# device time: 156750 ns/iter; 1.9953x vs baseline; 1.0006x over previous
import jax
import jax.numpy as jnp
from jax import lax
from jax.experimental import pallas as pl
from jax.experimental.pallas import tpu as pltpu

N_DEV = 4
M = 2048
N = 2048
M_CH = M // N_DEV
M_H = M_CH // 2
K_SUB = 4
M_S = M_H // K_SUB
N_HOP = N_DEV - 1


def kernel(x, w_mat):
    def body(x_hbm, w_hbm, out_hbm, x_ref, w_ref, y_ref, buf_cw, buf_ccw,
             send_cw, recv_cw, send_ccw, recv_ccw, store_sems, load_sems):
        my = lax.axis_index("i")
        left = (my - 1) % N_DEV
        right = (my + 1) % N_DEV

        w_load = pltpu.make_async_copy(w_hbm, w_ref, load_sems.at[0])
        w_load.start()
        x_loads = []
        for o in range(N_DEV):
            c = (my + o) % N_DEV
            r = pl.ds(c * M_CH, M_CH)
            cp = pltpu.make_async_copy(
                x_hbm.at[r, :], x_ref.at[r, :], load_sems.at[1 + o]
            )
            cp.start()
            x_loads.append(cp)

        barrier_sem = pltpu.get_barrier_semaphore()
        for nbr in (left, right):
            pl.semaphore_signal(
                barrier_sem, inc=1,
                device_id=(nbr,), device_id_type=pl.DeviceIdType.MESH,
            )
        pl.semaphore_wait(barrier_sem, 2)

        def a_rows(c, j):
            return pl.ds(c * M_CH + j * M_S, M_S)

        def b_rows(c, j):
            return pl.ds(c * M_CH + M_H + j * M_S, M_S)

        def ch_rows(c):
            return pl.ds(c * M_CH, M_CH)

        def rs_cw(s, j):
            return pltpu.make_async_remote_copy(
                src_ref=y_ref.at[a_rows((my - s) % N_DEV, j), :],
                dst_ref=buf_cw.at[s * K_SUB + j],
                send_sem=send_cw.at[s * K_SUB + j],
                recv_sem=recv_cw.at[s * K_SUB + j],
                device_id=(right,),
                device_id_type=pl.DeviceIdType.MESH,
            )

        def rs_ccw(s, j):
            return pltpu.make_async_remote_copy(
                src_ref=y_ref.at[b_rows((my + s) % N_DEV, j), :],
                dst_ref=buf_ccw.at[s * K_SUB + j],
                send_sem=send_ccw.at[s * K_SUB + j],
                recv_sem=recv_ccw.at[s * K_SUB + j],
                device_id=(left,),
                device_id_type=pl.DeviceIdType.MESH,
            )

        def ag_cw(t, j):
            c = (my + 1 - t) % N_DEV
            return pltpu.make_async_remote_copy(
                src_ref=y_ref.at[a_rows(c, j), :],
                dst_ref=y_ref.at[a_rows(c, j), :],
                send_sem=send_cw.at[(N_HOP + t) * K_SUB + j],
                recv_sem=recv_cw.at[(N_HOP + t) * K_SUB + j],
                device_id=(right,),
                device_id_type=pl.DeviceIdType.MESH,
            )

        def ag_ccw(t, j):
            c = (my - 1 + t) % N_DEV
            return pltpu.make_async_remote_copy(
                src_ref=y_ref.at[b_rows(c, j), :],
                dst_ref=y_ref.at[b_rows(c, j), :],
                send_sem=send_ccw.at[(N_HOP + t) * K_SUB + j],
                recv_sem=recv_ccw.at[(N_HOP + t) * K_SUB + j],
                device_id=(left,),
                device_id_type=pl.DeviceIdType.MESH,
            )

        pending_sends = []
        pending_stores = []
        n_stores = [0]

        def launch(d):
            d.start()
            pending_sends.append(d)
            return d

        def store(rows):
            cp = pltpu.make_async_copy(
                y_ref.at[rows, :], out_hbm.at[rows, :],
                store_sems.at[n_stores[0]],
            )
            n_stores[0] += 1
            cp.start()
            pending_stores.append(cp)

        w_load.wait()
        x_loads[0].wait()
        cw = [None] * K_SUB
        ccw = [None] * K_SUB
        for j in range(K_SUB):
            r = a_rows(my, j)
            y_ref[r, :] = jnp.dot(
                x_ref[r, :], w_ref[:, :], preferred_element_type=jnp.float32
            )
            cw[j] = launch(rs_cw(0, j))
            r = b_rows(my, j)
            y_ref[r, :] = jnp.dot(
                x_ref[r, :], w_ref[:, :], preferred_element_type=jnp.float32
            )
            ccw[j] = launch(rs_ccw(0, j))

        for o in range(1, N_DEV):
            c = (my + o) % N_DEV
            x_loads[o].wait()
            y_ref[ch_rows(c), :] = jnp.dot(
                x_ref[ch_rows(c), :], w_ref[:, :],
                preferred_element_type=jnp.float32,
            )

        for s in range(N_HOP - 1):
            for j in range(K_SUB):
                cw[j].wait_recv()
                y_ref[a_rows((my - s - 1) % N_DEV, j), :] += buf_cw[s * K_SUB + j]
                cw[j] = launch(rs_cw(s + 1, j))
                ccw[j].wait_recv()
                y_ref[b_rows((my + s + 1) % N_DEV, j), :] += buf_ccw[s * K_SUB + j]
                ccw[j] = launch(rs_ccw(s + 1, j))

        s = N_HOP - 1
        ca = (my + 1) % N_DEV
        cb = (my - 1) % N_DEV
        for j in range(K_SUB):
            cw[j].wait_recv()
            ya = y_ref[a_rows(ca, j), :] + buf_cw[s * K_SUB + j]
            y_ref[a_rows(ca, j), :] = ya * jax.nn.sigmoid(ya)
            cw[j] = launch(ag_cw(0, j))
            store(a_rows(ca, j))
            ccw[j].wait_recv()
            yb = y_ref[b_rows(cb, j), :] + buf_ccw[s * K_SUB + j]
            y_ref[b_rows(cb, j), :] = yb * jax.nn.sigmoid(yb)
            ccw[j] = launch(ag_ccw(0, j))
            store(b_rows(cb, j))

        for t in range(N_HOP):
            for j in range(K_SUB):
                cw[j].wait_recv()
                if t < N_HOP - 1:
                    cw[j] = launch(ag_cw(t + 1, j))
                store(a_rows((my - t) % N_DEV, j))
                ccw[j].wait_recv()
                if t < N_HOP - 1:
                    ccw[j] = launch(ag_ccw(t + 1, j))
                store(b_rows((my + t) % N_DEV, j))

        for d in pending_sends:
            d.wait_send()
        for cp in pending_stores:
            cp.wait()

    n_slots = 2 * N_HOP * K_SUB
    n_store_slots = 2 * N_DEV * K_SUB
    return pl.pallas_call(
        body,
        out_shape=jax.ShapeDtypeStruct((M, N), jnp.float32),
        in_specs=[
            pl.BlockSpec(memory_space=pl.MemorySpace.ANY),
            pl.BlockSpec(memory_space=pl.MemorySpace.ANY),
        ],
        out_specs=pl.BlockSpec(memory_space=pl.MemorySpace.ANY),
        scratch_shapes=[
            pltpu.VMEM((M, x.shape[1]), jnp.float32),
            pltpu.VMEM((x.shape[1], N), jnp.float32),
            pltpu.VMEM((M, N), jnp.float32),
            pltpu.VMEM((N_HOP * K_SUB, M_S, N), jnp.float32),
            pltpu.VMEM((N_HOP * K_SUB, M_S, N), jnp.float32),
            pltpu.SemaphoreType.DMA((n_slots,)),
            pltpu.SemaphoreType.DMA((n_slots,)),
            pltpu.SemaphoreType.DMA((n_slots,)),
            pltpu.SemaphoreType.DMA((n_slots,)),
            pltpu.SemaphoreType.DMA((n_store_slots,)),
            pltpu.SemaphoreType.DMA((1 + N_DEV,)),
        ],
        compiler_params=pltpu.CompilerParams(
            collective_id=0,
            vmem_limit_bytes=64 * 1024 * 1024,
        ),
    )(x, w_mat)
